# baseline (speedup 1.0000x reference)
import functools
import math

import jax
import jax.numpy as jnp
from jax import lax
from jax.experimental import pallas as pl
from jax.experimental.pallas import tpu as pltpu
import jax.experimental.pallas.tpu_sc as plsc

D_MODEL = 64
SCALE = math.sqrt(D_MODEL)

NC = 2
NS = 16
NW = NC * NS
LANES = 16

IDX_ROW = 128
CHUNK_ROWS = 4
CHUNK = CHUNK_ROWS * IDX_ROW


def _make_kernel(n_idx_rows: int):
    assert n_idx_rows % (NW * CHUNK_ROWS) == 0
    rows_per_w = n_idx_rows // NW
    n_chunks = rows_per_w // CHUNK_ROWS

    mesh = plsc.VectorSubcoreMesh(
        core_axis_name="c", subcore_axis_name="s",
        num_cores=NC, num_subcores=NS)

    @functools.partial(
        pl.kernel,
        out_type=jax.ShapeDtypeStruct((n_idx_rows * IDX_ROW, D_MODEL),
                                      jnp.float32),
        mesh=mesh,
        scratch_types=[
            pltpu.VMEM((CHUNK_ROWS, IDX_ROW), jnp.int32),
            pltpu.VMEM((CHUNK, D_MODEL), jnp.float32),
            pltpu.SemaphoreType.DMA,
        ],
        compiler_params=pltpu.CompilerParams(use_tc_tiling_on_sc=False),
    )
    def embed(idx_hbm, table_hbm, out_hbm, idx_v, rows_v, gsem):
        wid = lax.axis_index("s") * NC + lax.axis_index("c")
        base_row = wid * rows_per_w

        @pl.loop(0, n_chunks)
        def _chunk(g):
            r0 = base_row + g * CHUNK_ROWS
            pltpu.sync_copy(idx_hbm.at[pl.ds(r0, CHUNK_ROWS)], idx_v)
            for j in range(CHUNK_ROWS):
                pltpu.async_copy(
                    table_hbm.at[idx_v.at[j]],
                    rows_v.at[pl.ds(j * IDX_ROW, IDX_ROW)],
                    gsem)
            for j in range(CHUNK_ROWS):
                pltpu.make_async_copy(
                    table_hbm.at[idx_v.at[j]],
                    rows_v.at[pl.ds(j * IDX_ROW, IDX_ROW)],
                    gsem).wait()

            @plsc.parallel_loop(0, CHUNK, 1, unroll=4)
            def _scale(r):
                for c in range(D_MODEL // LANES):
                    sl = pl.ds(c * LANES, LANES)
                    rows_v[r, sl] = rows_v[r, sl] * SCALE

            pltpu.sync_copy(rows_v,
                            out_hbm.at[pl.ds(r0 * IDX_ROW, CHUNK)])

    return embed


def kernel(x, table):
    b, s = x.shape
    n = b * s
    idx2d = x.reshape(n // IDX_ROW, IDX_ROW).astype(jnp.int32)
    out = _make_kernel(n // IDX_ROW)(idx2d, table)
    return out.reshape(b, s, D_MODEL)


# speedup vs baseline: 1.0890x; 1.0890x over previous
import functools
import math

import jax
import jax.numpy as jnp
from jax import lax
from jax.experimental import pallas as pl
from jax.experimental.pallas import tpu as pltpu
import jax.experimental.pallas.tpu_sc as plsc

D_MODEL = 64
SCALE = math.sqrt(D_MODEL)

NC = 2
NS = 16
NW = NC * NS
LANES = 16

IDX_ROW = 128
CHUNK_ROWS = 4
CHUNK = CHUNK_ROWS * IDX_ROW
NBUF = 3


def _make_kernel(n_idx_rows: int):
    assert n_idx_rows % (NW * CHUNK_ROWS) == 0
    rows_per_w = n_idx_rows // NW
    n_chunks = rows_per_w // CHUNK_ROWS
    assert n_chunks >= 2 * NBUF

    mesh = plsc.VectorSubcoreMesh(
        core_axis_name="c", subcore_axis_name="s",
        num_cores=NC, num_subcores=NS)

    @functools.partial(
        pl.kernel,
        out_type=jax.ShapeDtypeStruct((n_idx_rows * IDX_ROW, D_MODEL),
                                      jnp.float32),
        mesh=mesh,
        scratch_types=[
            pltpu.VMEM((NBUF, CHUNK_ROWS, IDX_ROW), jnp.int32),
            pltpu.VMEM((NBUF, CHUNK, D_MODEL), jnp.float32),
            pltpu.SemaphoreType.DMA((NBUF,)),
            pltpu.SemaphoreType.DMA((NBUF,)),
        ],
        compiler_params=pltpu.CompilerParams(use_tc_tiling_on_sc=False),
    )
    def embed(idx_hbm, table_hbm, out_hbm, idx_v, rows_v, gsem, osem):
        wid = lax.axis_index("s") * NC + lax.axis_index("c")
        base_row = wid * rows_per_w

        def fire_gathers(i, b):
            r0 = base_row + i * CHUNK_ROWS
            pltpu.sync_copy(idx_hbm.at[pl.ds(r0, CHUNK_ROWS)], idx_v.at[b])
            for j in range(CHUNK_ROWS):
                pltpu.async_copy(
                    table_hbm.at[idx_v.at[b, j]],
                    rows_v.at[b, pl.ds(j * IDX_ROW, IDX_ROW)],
                    gsem.at[b])

        def wait_out(b):
            pltpu.make_async_copy(
                rows_v.at[b],
                out_hbm.at[pl.ds(0, CHUNK)],
                osem.at[b]).wait()

        def do_chunk(i, b, prefetch, wait_prev_out):
            bn = (b + 2) % NBUF
            for j in range(CHUNK_ROWS):
                pltpu.make_async_copy(
                    table_hbm.at[idx_v.at[b, j]],
                    rows_v.at[b, pl.ds(j * IDX_ROW, IDX_ROW)],
                    gsem.at[b]).wait()

            @plsc.parallel_loop(0, CHUNK, 1, unroll=4)
            def _scale(r):
                for c in range(D_MODEL // LANES):
                    sl = pl.ds(c * LANES, LANES)
                    rows_v[b, r, sl] = rows_v[b, r, sl] * SCALE

            r0 = base_row + i * CHUNK_ROWS
            pltpu.async_copy(
                rows_v.at[b],
                out_hbm.at[pl.ds(r0 * IDX_ROW, CHUNK)],
                osem.at[b])
            if prefetch:
                if wait_prev_out:
                    wait_out(bn)
                fire_gathers(i + 2, bn)

        fire_gathers(0, 0)
        fire_gathers(1, 1)
        do_chunk(0, 0, True, False)
        do_chunk(1, 1, True, True)
        do_chunk(2, 2, True, True)

        main_end = NBUF * ((n_chunks - 2 - 3) // NBUF) + 3

        @pl.loop(3, main_end, step=NBUF)
        def _group(g):
            for db in range(NBUF):
                do_chunk(g + db, db, True, True)

        for i in range(main_end, n_chunks):
            do_chunk(i, i % NBUF, i + 2 < n_chunks, True)

        for i in range(n_chunks - NBUF, n_chunks):
            wait_out(i % NBUF)

    return embed


def kernel(x, table):
    b, s = x.shape
    n = b * s
    idx2d = x.reshape(n // IDX_ROW, IDX_ROW).astype(jnp.int32)
    out = _make_kernel(n // IDX_ROW)(idx2d, table)
    return out.reshape(b, s, D_MODEL)
